# baseline (speedup 1.0000x reference)
import jax
import jax.numpy as jnp
from jax import lax
from jax.experimental import pallas as pl
from jax.experimental.pallas import tpu as pltpu
from jax.experimental.pallas import tpu_sc as plsc

N = 10000
E = 160000
NC, NS = 2, 16
NPAD = 10240

EW = 125
DR = 40
CW = 50
CPT = 200
PHASES = ((0, 56), (56, 48), (104, 48), (152, 48))
CPH = 56
NBUF = 4
RPT = NPAD // NS
RCH = 16

BN = 2048
GRID = NPAD // BN

_mesh = plsc.VectorSubcoreMesh(core_axis_name="c", subcore_axis_name="s")


def _deg_body(dst_hbm, ones_hbm, zeros_hbm, deg_hbm, idx_v, ones_v, z_v, acc):
    c = lax.axis_index("c")
    s = lax.axis_index("s")
    wid = s * NC + c
    pltpu.sync_copy(zeros_hbm, z_v)
    pltpu.sync_copy(z_v, acc.at[pl.ds(s * RPT, RPT)])
    pltpu.sync_copy(ones_hbm, ones_v)
    pltpu.sync_copy(dst_hbm.at[pl.ds(wid * DR, DR)], idx_v)
    plsc.subcore_barrier()

    def body(i, carry):
        pltpu.sync_copy(ones_v, acc.at[idx_v.at[i]], add=True)
        return carry

    lax.fori_loop(0, DR, body, 0)
    plsc.subcore_barrier()
    pltpu.sync_copy(acc.at[pl.ds(s * RPT, RPT)], z_v)
    pltpu.sync_copy(z_v, deg_hbm.at[c, 0, pl.ds(s * RPT, RPT)])


_deg_call = pl.kernel(
    _deg_body,
    out_type=jax.ShapeDtypeStruct((NC, 1, NPAD), jnp.float32),
    mesh=_mesh,
    scratch_types=[
        pltpu.VMEM((DR, EW), jnp.int32),
        pltpu.VMEM((EW,), jnp.float32),
        pltpu.VMEM((RPT,), jnp.float32),
        pltpu.VMEM_SHARED((NPAD,), jnp.float32),
    ],
)


def _agg_body(t_hbm, sidx_hbm, didx_hbm, zeros_hbm, out_hbm,
              sidx_v, didx_v, bufs, z_v, acc, gsem, tsem):
    c = lax.axis_index("c")
    s = lax.axis_index("s")
    pltpu.sync_copy(zeros_hbm, z_v)
    for j in range(RPT // RCH):
        pltpu.sync_copy(z_v, acc.at[pl.ds(s * RPT + j * RCH, RCH)])
    plsc.subcore_barrier()

    for off, rows in PHASES:
        base = s * CPT + off
        pltpu.sync_copy(sidx_hbm.at[c, pl.ds(base, rows)],
                        sidx_v.at[pl.ds(0, rows)])
        pltpu.sync_copy(didx_hbm.at[pl.ds(base, rows)],
                        didx_v.at[pl.ds(0, rows)])
        for k in range(NBUF - 1):
            pltpu.async_copy(t_hbm.at[sidx_v.at[k]], bufs[k], gsem[k])

        def body(jj, carry):
            for k in range(NBUF):
                i = NBUF * jj + k
                kp = (k + NBUF - 1) % NBUF

                @pl.when(i + NBUF - 1 < rows)
                def _():
                    @pl.when(i >= 1)
                    def _():
                        pltpu.make_async_copy(
                            bufs[kp], acc.at[didx_v.at[0]], tsem[kp]).wait()
                    pltpu.async_copy(t_hbm.at[sidx_v.at[i + NBUF - 1]],
                                     bufs[kp], gsem[kp])

                pltpu.make_async_copy(t_hbm.at[sidx_v.at[i]], bufs[k],
                                      gsem[k]).wait()
                pltpu.async_copy(bufs[k], acc.at[didx_v.at[i]], tsem[k],
                                 add=True)
            return carry

        lax.fori_loop(0, rows // NBUF, body, 0)
        for k in range(NBUF):
            pltpu.make_async_copy(bufs[k], acc.at[didx_v.at[0]],
                                  tsem[k]).wait()
    plsc.subcore_barrier()
    for j in range(RPT // RCH):
        pltpu.sync_copy(acc.at[pl.ds(s * RPT + j * RCH, RCH)], z_v)
        pltpu.sync_copy(z_v, out_hbm.at[c, pl.ds(s * RPT + j * RCH, RCH)])


_agg_call = pl.kernel(
    _agg_body,
    out_type=jax.ShapeDtypeStruct((NC, NPAD, 128), jnp.float32),
    mesh=_mesh,
    scratch_types=[
        pltpu.VMEM((CPH, CW), jnp.int32),
        pltpu.VMEM((CPH, CW), jnp.int32),
        tuple(pltpu.VMEM((CW, 128), jnp.float32) for _ in range(NBUF)),
        pltpu.VMEM((RCH, 128), jnp.float32),
        pltpu.VMEM_SHARED((NPAD, 128), jnp.float32),
        tuple(pltpu.SemaphoreType.DMA for _ in range(NBUF)),
        tuple(pltpu.SemaphoreType.DMA for _ in range(NBUF)),
    ],
)


def _dinv_of(deg_ref):
    d = deg_ref[0].reshape(BN) + deg_ref[1].reshape(BN) + 1.0
    return lax.rsqrt(d)[:, None]


def _scale_kernel(x_ref, deg_ref, o_ref):
    dinv = _dinv_of(deg_ref)
    o_ref[0] = x_ref[:, :128] * dinv
    o_ref[1] = x_ref[:, 128:] * dinv


def _combine(agg_ref, t_ref, dinv):
    return jnp.concatenate([agg_ref[0] + t_ref[0], agg_ref[1] + t_ref[1]],
                           axis=1) * dinv


def _mm_kernel(agg_ref, t_ref, deg_ref, w1_ref, b1_ref, w2_ref, o_ref):
    dinv = _dinv_of(deg_ref)
    y = _combine(agg_ref, t_ref, dinv)
    h = jnp.dot(y, w1_ref[...], preferred_element_type=jnp.float32) + b1_ref[...]
    h = jnp.maximum(h, 0.0)
    q = jnp.dot(h, w2_ref[...], preferred_element_type=jnp.float32)
    q = q * dinv
    o_ref[0] = q[:, :128]
    o_ref[1] = q[:, 128:]


def _post_kernel(agg_ref, t_ref, deg_ref, b_ref, o_ref):
    dinv = _dinv_of(deg_ref)
    o_ref[...] = _combine(agg_ref, t_ref, dinv) + b_ref[...]


def _split_spec():
    return pl.BlockSpec((NC, BN, 128), lambda i: (0, i, 0))


def _deg_spec():
    return pl.BlockSpec((NC, BN // 128, 128), lambda i: (0, i, 0))


def _full(shape):
    return pl.BlockSpec(shape, lambda i: tuple(0 for _ in shape))


_scale_call = pl.pallas_call(
    _scale_kernel,
    grid=GRID,
    in_specs=[pl.BlockSpec((BN, 256), lambda i: (i, 0)), _deg_spec()],
    out_specs=_split_spec(),
    out_shape=jax.ShapeDtypeStruct((NC, NPAD, 128), jnp.float32),
)

_mm_call = pl.pallas_call(
    _mm_kernel,
    grid=GRID,
    in_specs=[_split_spec(), _split_spec(), _deg_spec(),
              _full((256, 512)), _full((1, 512)), _full((512, 256))],
    out_specs=_split_spec(),
    out_shape=jax.ShapeDtypeStruct((NC, NPAD, 128), jnp.float32),
)

_post_call = pl.pallas_call(
    _post_kernel,
    grid=GRID,
    in_specs=[_split_spec(), _split_spec(), _deg_spec(), _full((1, 256))],
    out_specs=pl.BlockSpec((BN, 256), lambda i: (i, 0)),
    out_shape=jax.ShapeDtypeStruct((N, 256), jnp.float32),
)


def kernel(x, edge_index, W1, b1, W2, b2):
    ei = edge_index.astype(jnp.int32)
    src = ei[0].reshape(E // CW, CW)
    src_stack = jnp.stack([src, src + NPAD])
    dst2d = ei[1].reshape(E // CW, CW)
    dstdeg = ei[1].reshape(E // EW, EW)
    ones_w = jnp.ones((EW,), jnp.float32)
    zeros1d = jnp.zeros((RPT,), jnp.float32)
    zeros2d = jnp.zeros((RCH, 128), jnp.float32)

    deg2 = _deg_call(dstdeg, ones_w, zeros1d)
    deg2p = deg2.reshape(NC, NPAD // 128, 128)

    t1 = _scale_call(x, deg2p)
    agg1 = _agg_call(t1.reshape(NC * NPAD, 128), src_stack, dst2d, zeros2d)
    t2 = _mm_call(agg1, t1, deg2p, W1, b1.reshape(1, -1), W2)
    agg2 = _agg_call(t2.reshape(NC * NPAD, 128), src_stack, dst2d, zeros2d)
    return _post_call(agg2, t2, deg2p, b2.reshape(1, -1))


# speedup vs baseline: 1.0045x; 1.0045x over previous
import jax
import jax.numpy as jnp
from jax import lax
from jax.experimental import pallas as pl
from jax.experimental.pallas import tpu as pltpu
from jax.experimental.pallas import tpu_sc as plsc

N = 10000
E = 160000
NC, NS = 2, 16
NPAD = 10240

EW = 125
DR = 40
CW = 50
CPT = 200
PHASES = ((0, 56), (56, 48), (104, 48), (152, 48))
CPH = 56
NBUF = 4
RPT = NPAD // NS
RCH = 16

BN = 2048
GRID = NPAD // BN

_mesh = plsc.VectorSubcoreMesh(core_axis_name="c", subcore_axis_name="s")


def _deg_body(dst_hbm, ones_hbm, zeros_hbm, deg_hbm, idx_v, ones_v, z_v, acc):
    c = lax.axis_index("c")
    s = lax.axis_index("s")
    wid = s * NC + c
    pltpu.sync_copy(zeros_hbm, z_v)
    pltpu.sync_copy(z_v, acc.at[pl.ds(s * RPT, RPT)])
    pltpu.sync_copy(ones_hbm, ones_v)
    pltpu.sync_copy(dst_hbm.at[pl.ds(wid * DR, DR)], idx_v)
    plsc.subcore_barrier()

    def body(i, carry):
        pltpu.sync_copy(ones_v, acc.at[idx_v.at[i]], add=True)
        return carry

    lax.fori_loop(0, DR, body, 0)
    plsc.subcore_barrier()
    pltpu.sync_copy(acc.at[pl.ds(s * RPT, RPT)], z_v)
    pltpu.sync_copy(z_v, deg_hbm.at[c, 0, pl.ds(s * RPT, RPT)])


_deg_call = pl.kernel(
    _deg_body,
    out_type=jax.ShapeDtypeStruct((NC, 1, NPAD), jnp.float32),
    mesh=_mesh,
    scratch_types=[
        pltpu.VMEM((DR, EW), jnp.int32),
        pltpu.VMEM((EW,), jnp.float32),
        pltpu.VMEM((RPT,), jnp.float32),
        pltpu.VMEM_SHARED((NPAD,), jnp.float32),
    ],
)


def _agg_body(t_hbm, sidx_hbm, didx_hbm, zeros_hbm, out_hbm,
              sidx_v, didx_v, bufs, z_v, acc, gsem, tsem):
    c = lax.axis_index("c")
    s = lax.axis_index("s")
    off0, rows0 = PHASES[0]
    pltpu.sync_copy(sidx_hbm.at[c, pl.ds(s * CPT + off0, rows0)],
                    sidx_v.at[pl.ds(0, rows0)])
    pltpu.sync_copy(didx_hbm.at[pl.ds(s * CPT + off0, rows0)],
                    didx_v.at[pl.ds(0, rows0)])
    for k in range(NBUF - 1):
        pltpu.async_copy(t_hbm.at[sidx_v.at[k]], bufs[k], gsem[k])
    pltpu.sync_copy(zeros_hbm, z_v)
    for j in range(RPT // RCH):
        pltpu.sync_copy(z_v, acc.at[pl.ds(s * RPT + j * RCH, RCH)])
    plsc.subcore_barrier()

    for pnum, (off, rows) in enumerate(PHASES):
        if pnum > 0:
            base = s * CPT + off
            pltpu.sync_copy(sidx_hbm.at[c, pl.ds(base, rows)],
                            sidx_v.at[pl.ds(0, rows)])
            pltpu.sync_copy(didx_hbm.at[pl.ds(base, rows)],
                            didx_v.at[pl.ds(0, rows)])
            for k in range(NBUF - 1):
                pltpu.async_copy(t_hbm.at[sidx_v.at[k]], bufs[k], gsem[k])

        def body(jj, carry):
            for k in range(NBUF):
                i = NBUF * jj + k
                kp = (k + NBUF - 1) % NBUF

                @pl.when(i + NBUF - 1 < rows)
                def _():
                    @pl.when(i >= 1)
                    def _():
                        pltpu.make_async_copy(
                            bufs[kp], acc.at[didx_v.at[0]], tsem[kp]).wait()
                    pltpu.async_copy(t_hbm.at[sidx_v.at[i + NBUF - 1]],
                                     bufs[kp], gsem[kp])

                pltpu.make_async_copy(t_hbm.at[sidx_v.at[i]], bufs[k],
                                      gsem[k]).wait()
                pltpu.async_copy(bufs[k], acc.at[didx_v.at[i]], tsem[k],
                                 add=True)
            return carry

        lax.fori_loop(0, rows // NBUF, body, 0)
        for k in range(NBUF):
            pltpu.make_async_copy(bufs[k], acc.at[didx_v.at[0]],
                                  tsem[k]).wait()
    plsc.subcore_barrier()
    for j in range(RPT // RCH):
        pltpu.sync_copy(acc.at[pl.ds(s * RPT + j * RCH, RCH)], z_v)
        pltpu.sync_copy(z_v, out_hbm.at[c, pl.ds(s * RPT + j * RCH, RCH)])


_agg_call = pl.kernel(
    _agg_body,
    out_type=jax.ShapeDtypeStruct((NC, NPAD, 128), jnp.float32),
    mesh=_mesh,
    scratch_types=[
        pltpu.VMEM((CPH, CW), jnp.int32),
        pltpu.VMEM((CPH, CW), jnp.int32),
        tuple(pltpu.VMEM((CW, 128), jnp.float32) for _ in range(NBUF)),
        pltpu.VMEM((RCH, 128), jnp.float32),
        pltpu.VMEM_SHARED((NPAD, 128), jnp.float32),
        tuple(pltpu.SemaphoreType.DMA for _ in range(NBUF)),
        tuple(pltpu.SemaphoreType.DMA for _ in range(NBUF)),
    ],
)


def _dinv_of(deg_ref):
    d = deg_ref[0].reshape(BN) + deg_ref[1].reshape(BN) + 1.0
    return lax.rsqrt(d)[:, None]


def _scale_kernel(x_ref, deg_ref, o_ref):
    dinv = _dinv_of(deg_ref)
    o_ref[0] = x_ref[:, :128] * dinv
    o_ref[1] = x_ref[:, 128:] * dinv


def _combine(agg_ref, t_ref, dinv):
    return jnp.concatenate([agg_ref[0] + t_ref[0], agg_ref[1] + t_ref[1]],
                           axis=1) * dinv


def _mm_kernel(agg_ref, t_ref, deg_ref, w1_ref, b1_ref, w2_ref, o_ref):
    dinv = _dinv_of(deg_ref)
    y = _combine(agg_ref, t_ref, dinv)
    h = jnp.dot(y, w1_ref[...], preferred_element_type=jnp.float32) + b1_ref[...]
    h = jnp.maximum(h, 0.0)
    q = jnp.dot(h, w2_ref[...], preferred_element_type=jnp.float32)
    q = q * dinv
    o_ref[0] = q[:, :128]
    o_ref[1] = q[:, 128:]


def _post_kernel(agg_ref, t_ref, deg_ref, b_ref, o_ref):
    dinv = _dinv_of(deg_ref)
    o_ref[...] = _combine(agg_ref, t_ref, dinv) + b_ref[...]


def _split_spec():
    return pl.BlockSpec((NC, BN, 128), lambda i: (0, i, 0))


def _deg_spec():
    return pl.BlockSpec((NC, BN // 128, 128), lambda i: (0, i, 0))


def _full(shape):
    return pl.BlockSpec(shape, lambda i: tuple(0 for _ in shape))


_scale_call = pl.pallas_call(
    _scale_kernel,
    grid=GRID,
    in_specs=[pl.BlockSpec((BN, 256), lambda i: (i, 0)), _deg_spec()],
    out_specs=_split_spec(),
    out_shape=jax.ShapeDtypeStruct((NC, NPAD, 128), jnp.float32),
)

_mm_call = pl.pallas_call(
    _mm_kernel,
    grid=GRID,
    in_specs=[_split_spec(), _split_spec(), _deg_spec(),
              _full((256, 512)), _full((1, 512)), _full((512, 256))],
    out_specs=_split_spec(),
    out_shape=jax.ShapeDtypeStruct((NC, NPAD, 128), jnp.float32),
)

_post_call = pl.pallas_call(
    _post_kernel,
    grid=GRID,
    in_specs=[_split_spec(), _split_spec(), _deg_spec(), _full((1, 256))],
    out_specs=pl.BlockSpec((BN, 256), lambda i: (i, 0)),
    out_shape=jax.ShapeDtypeStruct((N, 256), jnp.float32),
)


def kernel(x, edge_index, W1, b1, W2, b2):
    ei = edge_index.astype(jnp.int32)
    src = ei[0].reshape(E // CW, CW)
    src_stack = jnp.stack([src, src + NPAD])
    dst2d = ei[1].reshape(E // CW, CW)
    dstdeg = ei[1].reshape(E // EW, EW)
    ones_w = jnp.ones((EW,), jnp.float32)
    zeros1d = jnp.zeros((RPT,), jnp.float32)
    zeros2d = jnp.zeros((RCH, 128), jnp.float32)

    deg2 = _deg_call(dstdeg, ones_w, zeros1d)
    deg2p = deg2.reshape(NC, NPAD // 128, 128)

    t1 = _scale_call(x, deg2p)
    agg1 = _agg_call(t1.reshape(NC * NPAD, 128), src_stack, dst2d, zeros2d)
    t2 = _mm_call(agg1, t1, deg2p, W1, b1.reshape(1, -1), W2)
    agg2 = _agg_call(t2.reshape(NC * NPAD, 128), src_stack, dst2d, zeros2d)
    return _post_call(agg2, t2, deg2p, b2.reshape(1, -1))


# speedup vs baseline: 1.0452x; 1.0405x over previous
import jax
import jax.numpy as jnp
from jax import lax
from jax.experimental import pallas as pl
from jax.experimental.pallas import tpu as pltpu
from jax.experimental.pallas import tpu_sc as plsc

N = 10000
E = 160000
NC, NS = 2, 16
NPAD = 10240

EW = 125
DR = 40
CW = 50
CPT = 200
PHASES = ((0, 56), (56, 48), (104, 48), (152, 48))
CPH = 56
NBUF = 4
RPT = NPAD // NS
RCH = 32

BN = 2048
GRID = NPAD // BN

_mesh = plsc.VectorSubcoreMesh(core_axis_name="c", subcore_axis_name="s")


def _deg_body(dst_hbm, ones_hbm, zeros_hbm, deg_hbm, idx_v, ones_v, z_v, acc,
              sem):
    c = lax.axis_index("c")
    s = lax.axis_index("s")
    wid = s * NC + c
    pltpu.sync_copy(zeros_hbm, z_v)
    pltpu.sync_copy(z_v, acc.at[pl.ds(s * RPT, RPT)])
    pltpu.sync_copy(ones_hbm, ones_v)
    pltpu.sync_copy(dst_hbm.at[pl.ds(wid * DR, DR)], idx_v)
    plsc.subcore_barrier()

    def body(i, carry):
        pltpu.async_copy(ones_v, acc.at[idx_v.at[i]], sem, add=True)
        return carry

    lax.fori_loop(0, DR, body, 0)

    def drain(i, carry):
        pltpu.make_async_copy(ones_v, acc.at[idx_v.at[0]], sem).wait()
        return carry

    lax.fori_loop(0, DR, drain, 0)
    plsc.subcore_barrier()
    pltpu.sync_copy(acc.at[pl.ds(s * RPT, RPT)], z_v)
    pltpu.sync_copy(z_v, deg_hbm.at[c, 0, pl.ds(s * RPT, RPT)])


_deg_call = pl.kernel(
    _deg_body,
    out_type=jax.ShapeDtypeStruct((NC, 1, NPAD), jnp.float32),
    mesh=_mesh,
    scratch_types=[
        pltpu.VMEM((DR, EW), jnp.int32),
        pltpu.VMEM((EW,), jnp.float32),
        pltpu.VMEM((RPT,), jnp.float32),
        pltpu.VMEM_SHARED((NPAD,), jnp.float32),
        pltpu.SemaphoreType.DMA,
    ],
)


def _agg_body(t_hbm, sidx_hbm, didx_hbm, zeros_hbm, out_hbm,
              sidx_v, didx_v, bufs, z_v, acc, gsem, tsem):
    c = lax.axis_index("c")
    s = lax.axis_index("s")
    off0, rows0 = PHASES[0]
    pltpu.sync_copy(sidx_hbm.at[c, pl.ds(s * CPT + off0, rows0)],
                    sidx_v.at[pl.ds(0, rows0)])
    pltpu.sync_copy(didx_hbm.at[pl.ds(s * CPT + off0, rows0)],
                    didx_v.at[pl.ds(0, rows0)])
    for k in range(NBUF - 1):
        pltpu.async_copy(t_hbm.at[sidx_v.at[k]], bufs[k], gsem[k])
    pltpu.sync_copy(zeros_hbm, z_v)
    for j in range(RPT // RCH):
        pltpu.sync_copy(z_v, acc.at[pl.ds(s * RPT + j * RCH, RCH)])
    plsc.subcore_barrier()

    for pnum, (off, rows) in enumerate(PHASES):
        if pnum > 0:
            base = s * CPT + off
            pltpu.sync_copy(sidx_hbm.at[c, pl.ds(base, rows)],
                            sidx_v.at[pl.ds(0, rows)])
            pltpu.sync_copy(didx_hbm.at[pl.ds(base, rows)],
                            didx_v.at[pl.ds(0, rows)])
            for k in range(NBUF - 1):
                pltpu.async_copy(t_hbm.at[sidx_v.at[k]], bufs[k], gsem[k])

        def body(jj, carry):
            for k in range(NBUF):
                i = NBUF * jj + k
                kp = (k + NBUF - 1) % NBUF

                @pl.when(i + NBUF - 1 < rows)
                def _():
                    @pl.when(i >= 1)
                    def _():
                        pltpu.make_async_copy(
                            bufs[kp], acc.at[didx_v.at[0]], tsem[kp]).wait()
                    pltpu.async_copy(t_hbm.at[sidx_v.at[i + NBUF - 1]],
                                     bufs[kp], gsem[kp])

                pltpu.make_async_copy(t_hbm.at[sidx_v.at[i]], bufs[k],
                                      gsem[k]).wait()
                pltpu.async_copy(bufs[k], acc.at[didx_v.at[i]], tsem[k],
                                 add=True)
            return carry

        lax.fori_loop(0, rows // NBUF, body, 0)
        for k in range(NBUF):
            pltpu.make_async_copy(bufs[k], acc.at[didx_v.at[0]],
                                  tsem[k]).wait()
    plsc.subcore_barrier()
    for j in range(RPT // RCH):
        pltpu.sync_copy(acc.at[pl.ds(s * RPT + j * RCH, RCH)], z_v)
        pltpu.sync_copy(z_v, out_hbm.at[c, pl.ds(s * RPT + j * RCH, RCH)])


_agg_call = pl.kernel(
    _agg_body,
    out_type=jax.ShapeDtypeStruct((NC, NPAD, 128), jnp.float32),
    mesh=_mesh,
    scratch_types=[
        pltpu.VMEM((CPH, CW), jnp.int32),
        pltpu.VMEM((CPH, CW), jnp.int32),
        tuple(pltpu.VMEM((CW, 128), jnp.float32) for _ in range(NBUF)),
        pltpu.VMEM((RCH, 128), jnp.float32),
        pltpu.VMEM_SHARED((NPAD, 128), jnp.float32),
        tuple(pltpu.SemaphoreType.DMA for _ in range(NBUF)),
        tuple(pltpu.SemaphoreType.DMA for _ in range(NBUF)),
    ],
)


def _dinv_of(deg_ref):
    d = deg_ref[0].reshape(BN) + deg_ref[1].reshape(BN) + 1.0
    return lax.rsqrt(d)[:, None]


def _scale_kernel(x_ref, deg_ref, o_ref):
    dinv = _dinv_of(deg_ref)
    o_ref[0] = x_ref[:, :128] * dinv
    o_ref[1] = x_ref[:, 128:] * dinv


def _combine(agg_ref, t_ref, dinv):
    return jnp.concatenate([agg_ref[0] + t_ref[0], agg_ref[1] + t_ref[1]],
                           axis=1) * dinv


def _mm_kernel(agg_ref, t_ref, deg_ref, w1_ref, b1_ref, w2_ref, o_ref):
    dinv = _dinv_of(deg_ref)
    y = _combine(agg_ref, t_ref, dinv)
    h = jnp.dot(y, w1_ref[...], preferred_element_type=jnp.float32) + b1_ref[...]
    h = jnp.maximum(h, 0.0)
    q = jnp.dot(h, w2_ref[...], preferred_element_type=jnp.float32)
    q = q * dinv
    o_ref[0] = q[:, :128]
    o_ref[1] = q[:, 128:]


def _post_kernel(agg_ref, t_ref, deg_ref, b_ref, o_ref):
    dinv = _dinv_of(deg_ref)
    o_ref[...] = _combine(agg_ref, t_ref, dinv) + b_ref[...]


def _split_spec():
    return pl.BlockSpec((NC, BN, 128), lambda i: (0, i, 0))


def _deg_spec():
    return pl.BlockSpec((NC, BN // 128, 128), lambda i: (0, i, 0))


def _full(shape):
    return pl.BlockSpec(shape, lambda i: tuple(0 for _ in shape))


_scale_call = pl.pallas_call(
    _scale_kernel,
    grid=GRID,
    in_specs=[pl.BlockSpec((BN, 256), lambda i: (i, 0)), _deg_spec()],
    out_specs=_split_spec(),
    out_shape=jax.ShapeDtypeStruct((NC, NPAD, 128), jnp.float32),
)

_mm_call = pl.pallas_call(
    _mm_kernel,
    grid=GRID,
    in_specs=[_split_spec(), _split_spec(), _deg_spec(),
              _full((256, 512)), _full((1, 512)), _full((512, 256))],
    out_specs=_split_spec(),
    out_shape=jax.ShapeDtypeStruct((NC, NPAD, 128), jnp.float32),
)

_post_call = pl.pallas_call(
    _post_kernel,
    grid=GRID,
    in_specs=[_split_spec(), _split_spec(), _deg_spec(), _full((1, 256))],
    out_specs=pl.BlockSpec((BN, 256), lambda i: (i, 0)),
    out_shape=jax.ShapeDtypeStruct((N, 256), jnp.float32),
)


def kernel(x, edge_index, W1, b1, W2, b2):
    ei = edge_index.astype(jnp.int32)
    src = ei[0].reshape(E // CW, CW)
    src_stack = jnp.stack([src, src + NPAD])
    dst2d = ei[1].reshape(E // CW, CW)
    dstdeg = ei[1].reshape(E // EW, EW)
    ones_w = jnp.ones((EW,), jnp.float32)
    zeros1d = jnp.zeros((RPT,), jnp.float32)
    zeros2d = jnp.zeros((RCH, 128), jnp.float32)

    deg2 = _deg_call(dstdeg, ones_w, zeros1d)
    deg2p = deg2.reshape(NC, NPAD // 128, 128)

    t1 = _scale_call(x, deg2p)
    agg1 = _agg_call(t1.reshape(NC * NPAD, 128), src_stack, dst2d, zeros2d)
    t2 = _mm_call(agg1, t1, deg2p, W1, b1.reshape(1, -1), W2)
    agg2 = _agg_call(t2.reshape(NC * NPAD, 128), src_stack, dst2d, zeros2d)
    return _post_call(agg2, t2, deg2p, b2.reshape(1, -1))


# speedup vs baseline: 1.0575x; 1.0118x over previous
import jax
import jax.numpy as jnp
from jax import lax
from jax.experimental import pallas as pl
from jax.experimental.pallas import tpu as pltpu
from jax.experimental.pallas import tpu_sc as plsc

N = 10000
E = 160000
NC, NS = 2, 16
NPAD = 10240

EW = 125
DR = 40
CW = 50
CPT = 200
PHASES = ((0, 56), (56, 48), (104, 48), (152, 48))
CPH = 56
NBUF = 4
RPT = NPAD // NS
RCH = 32

BN = 2048
GRID = NPAD // BN

_mesh = plsc.VectorSubcoreMesh(core_axis_name="c", subcore_axis_name="s")


def _deg_body(dst_hbm, ones_hbm, zeros_hbm, deg_hbm, idx_v, ones_v, z_v, acc,
              sem):
    c = lax.axis_index("c")
    s = lax.axis_index("s")
    wid = s * NC + c
    pltpu.sync_copy(zeros_hbm, z_v)
    pltpu.sync_copy(z_v, acc.at[pl.ds(s * RPT, RPT)])
    pltpu.sync_copy(ones_hbm, ones_v)
    pltpu.sync_copy(dst_hbm.at[pl.ds(wid * DR, DR)], idx_v)
    plsc.subcore_barrier()

    def body(i, carry):
        pltpu.async_copy(ones_v, acc.at[idx_v.at[i]], sem, add=True)
        return carry

    lax.fori_loop(0, DR, body, 0)

    def drain(i, carry):
        pltpu.make_async_copy(ones_v, acc.at[idx_v.at[0]], sem).wait()
        return carry

    lax.fori_loop(0, DR, drain, 0)
    plsc.subcore_barrier()
    pltpu.sync_copy(acc.at[pl.ds(s * RPT, RPT)], z_v)
    pltpu.sync_copy(z_v, deg_hbm.at[c, 0, pl.ds(s * RPT, RPT)])


_deg_call = pl.kernel(
    _deg_body,
    out_type=jax.ShapeDtypeStruct((NC, 1, NPAD), jnp.float32),
    mesh=_mesh,
    scratch_types=[
        pltpu.VMEM((DR, EW), jnp.int32),
        pltpu.VMEM((EW,), jnp.float32),
        pltpu.VMEM((RPT,), jnp.float32),
        pltpu.VMEM_SHARED((NPAD,), jnp.float32),
        pltpu.SemaphoreType.DMA,
    ],
)


def _agg_body(t_hbm, sidx_hbm, didx_hbm, zeros_hbm, out_hbm,
              sidx_v, didx_v, bufs, acc, gsem, tsem):
    c = lax.axis_index("c")
    s = lax.axis_index("s")
    off0, rows0 = PHASES[0]
    pltpu.sync_copy(sidx_hbm.at[c, pl.ds(s * CPT + off0, rows0)],
                    sidx_v.at[pl.ds(0, rows0)])
    pltpu.sync_copy(didx_hbm.at[pl.ds(s * CPT + off0, rows0)],
                    didx_v.at[pl.ds(0, rows0)])
    for k in range(NBUF - 1):
        pltpu.async_copy(t_hbm.at[sidx_v.at[k]], bufs[k], gsem[k])
    pltpu.sync_copy(zeros_hbm, acc.at[pl.ds(s * RPT, RPT)])
    plsc.subcore_barrier()

    for pnum, (off, rows) in enumerate(PHASES):
        if pnum > 0:
            base = s * CPT + off
            pltpu.sync_copy(sidx_hbm.at[c, pl.ds(base, rows)],
                            sidx_v.at[pl.ds(0, rows)])
            pltpu.sync_copy(didx_hbm.at[pl.ds(base, rows)],
                            didx_v.at[pl.ds(0, rows)])
            for k in range(NBUF - 1):
                pltpu.async_copy(t_hbm.at[sidx_v.at[k]], bufs[k], gsem[k])

        def body(jj, carry):
            for k in range(NBUF):
                i = NBUF * jj + k
                kp = (k + NBUF - 1) % NBUF

                @pl.when(i + NBUF - 1 < rows)
                def _():
                    @pl.when(i >= 1)
                    def _():
                        pltpu.make_async_copy(
                            bufs[kp], acc.at[didx_v.at[0]], tsem[kp]).wait()
                    pltpu.async_copy(t_hbm.at[sidx_v.at[i + NBUF - 1]],
                                     bufs[kp], gsem[kp])

                pltpu.make_async_copy(t_hbm.at[sidx_v.at[i]], bufs[k],
                                      gsem[k]).wait()
                pltpu.async_copy(bufs[k], acc.at[didx_v.at[i]], tsem[k],
                                 add=True)
            return carry

        lax.fori_loop(0, rows // NBUF, body, 0)
        for k in range(NBUF):
            pltpu.make_async_copy(bufs[k], acc.at[didx_v.at[0]],
                                  tsem[k]).wait()
    plsc.subcore_barrier()
    pltpu.sync_copy(acc.at[pl.ds(s * RPT, RPT)],
                    out_hbm.at[c, pl.ds(s * RPT, RPT)])


_agg_call = pl.kernel(
    _agg_body,
    out_type=jax.ShapeDtypeStruct((NC, NPAD, 128), jnp.float32),
    mesh=_mesh,
    scratch_types=[
        pltpu.VMEM((CPH, CW), jnp.int32),
        pltpu.VMEM((CPH, CW), jnp.int32),
        tuple(pltpu.VMEM((CW, 128), jnp.float32) for _ in range(NBUF)),
        pltpu.VMEM_SHARED((NPAD, 128), jnp.float32),
        tuple(pltpu.SemaphoreType.DMA for _ in range(NBUF)),
        tuple(pltpu.SemaphoreType.DMA for _ in range(NBUF)),
    ],
)


def _dinv_of(deg_ref):
    d = deg_ref[0].reshape(BN) + deg_ref[1].reshape(BN) + 1.0
    return lax.rsqrt(d)[:, None]


def _scale_kernel(x_ref, deg_ref, o_ref):
    dinv = _dinv_of(deg_ref)
    o_ref[0] = x_ref[:, :128] * dinv
    o_ref[1] = x_ref[:, 128:] * dinv


def _combine(agg_ref, t_ref, dinv):
    return jnp.concatenate([agg_ref[0] + t_ref[0], agg_ref[1] + t_ref[1]],
                           axis=1) * dinv


def _mm_kernel(agg_ref, t_ref, deg_ref, w1_ref, b1_ref, w2_ref, o_ref):
    dinv = _dinv_of(deg_ref)
    y = _combine(agg_ref, t_ref, dinv)
    h = jnp.dot(y, w1_ref[...], preferred_element_type=jnp.float32) + b1_ref[...]
    h = jnp.maximum(h, 0.0)
    q = jnp.dot(h, w2_ref[...], preferred_element_type=jnp.float32)
    q = q * dinv
    o_ref[0] = q[:, :128]
    o_ref[1] = q[:, 128:]


def _post_kernel(agg_ref, t_ref, deg_ref, b_ref, o_ref):
    dinv = _dinv_of(deg_ref)
    o_ref[...] = _combine(agg_ref, t_ref, dinv) + b_ref[...]


def _split_spec():
    return pl.BlockSpec((NC, BN, 128), lambda i: (0, i, 0))


def _deg_spec():
    return pl.BlockSpec((NC, BN // 128, 128), lambda i: (0, i, 0))


def _full(shape):
    return pl.BlockSpec(shape, lambda i: tuple(0 for _ in shape))


_scale_call = pl.pallas_call(
    _scale_kernel,
    grid=GRID,
    in_specs=[pl.BlockSpec((BN, 256), lambda i: (i, 0)), _deg_spec()],
    out_specs=_split_spec(),
    out_shape=jax.ShapeDtypeStruct((NC, NPAD, 128), jnp.float32),
)

_mm_call = pl.pallas_call(
    _mm_kernel,
    grid=GRID,
    in_specs=[_split_spec(), _split_spec(), _deg_spec(),
              _full((256, 512)), _full((1, 512)), _full((512, 256))],
    out_specs=_split_spec(),
    out_shape=jax.ShapeDtypeStruct((NC, NPAD, 128), jnp.float32),
)

_post_call = pl.pallas_call(
    _post_kernel,
    grid=GRID,
    in_specs=[_split_spec(), _split_spec(), _deg_spec(), _full((1, 256))],
    out_specs=pl.BlockSpec((BN, 256), lambda i: (i, 0)),
    out_shape=jax.ShapeDtypeStruct((N, 256), jnp.float32),
)


def kernel(x, edge_index, W1, b1, W2, b2):
    ei = edge_index.astype(jnp.int32)
    src = ei[0].reshape(E // CW, CW)
    src_stack = jnp.stack([src, src + NPAD])
    dst2d = ei[1].reshape(E // CW, CW)
    dstdeg = ei[1].reshape(E // EW, EW)
    ones_w = jnp.ones((EW,), jnp.float32)
    zeros1d = jnp.zeros((RPT,), jnp.float32)
    zeros2d = jnp.zeros((RPT, 128), jnp.float32)

    deg2 = _deg_call(dstdeg, ones_w, zeros1d)
    deg2p = deg2.reshape(NC, NPAD // 128, 128)

    t1 = _scale_call(x, deg2p)
    agg1 = _agg_call(t1.reshape(NC * NPAD, 128), src_stack, dst2d, zeros2d)
    t2 = _mm_call(agg1, t1, deg2p, W1, b1.reshape(1, -1), W2)
    agg2 = _agg_call(t2.reshape(NC * NPAD, 128), src_stack, dst2d, zeros2d)
    return _post_call(agg2, t2, deg2p, b2.reshape(1, -1))


# speedup vs baseline: 1.0617x; 1.0039x over previous
import jax
import jax.numpy as jnp
from jax import lax
from jax.experimental import pallas as pl
from jax.experimental.pallas import tpu as pltpu
from jax.experimental.pallas import tpu_sc as plsc

N = 10000
E = 160000
NC, NS = 2, 16
NPAD = 10240

EW = 125
DR = 40
CW = 50
CPT = 200
PHASES = ((0, 56), (56, 48), (104, 48), (152, 48))
CPH = 56
NBUF = 4
RPT = NPAD // NS
RCH = 32

BN = 2048
GRID = NPAD // BN

_mesh = plsc.VectorSubcoreMesh(core_axis_name="c", subcore_axis_name="s")


def _deg_body(dst_hbm, ones_hbm, zeros_hbm, deg_hbm, idx_v, ones_v, acc,
              sem):
    c = lax.axis_index("c")
    s = lax.axis_index("s")
    wid = s * NC + c
    pltpu.sync_copy(zeros_hbm, acc.at[pl.ds(s * RPT, RPT)])
    pltpu.sync_copy(ones_hbm, ones_v)
    pltpu.sync_copy(dst_hbm.at[pl.ds(wid * DR, DR)], idx_v)
    plsc.subcore_barrier()

    def body(i, carry):
        pltpu.async_copy(ones_v, acc.at[idx_v.at[i]], sem, add=True)
        return carry

    lax.fori_loop(0, DR, body, 0)

    def drain(i, carry):
        pltpu.make_async_copy(ones_v, acc.at[idx_v.at[0]], sem).wait()
        return carry

    lax.fori_loop(0, DR, drain, 0)
    plsc.subcore_barrier()
    pltpu.sync_copy(acc.at[pl.ds(s * RPT, RPT)],
                    deg_hbm.at[c, 0, pl.ds(s * RPT, RPT)])


_deg_call = pl.kernel(
    _deg_body,
    out_type=jax.ShapeDtypeStruct((NC, 1, NPAD), jnp.float32),
    mesh=_mesh,
    scratch_types=[
        pltpu.VMEM((DR, EW), jnp.int32),
        pltpu.VMEM((EW,), jnp.float32),
        pltpu.VMEM_SHARED((NPAD,), jnp.float32),
        pltpu.SemaphoreType.DMA,
    ],
)


def _agg_body(t_hbm, sidx_hbm, didx_hbm, zeros_hbm, out_hbm,
              sidx_v, didx_v, bufs, acc, gsem, tsem):
    c = lax.axis_index("c")
    s = lax.axis_index("s")
    off0, rows0 = PHASES[0]
    pltpu.sync_copy(sidx_hbm.at[c, pl.ds(s * CPT + off0, rows0)],
                    sidx_v.at[pl.ds(0, rows0)])
    pltpu.sync_copy(didx_hbm.at[pl.ds(s * CPT + off0, rows0)],
                    didx_v.at[pl.ds(0, rows0)])
    for k in range(NBUF - 1):
        pltpu.async_copy(t_hbm.at[sidx_v.at[k]], bufs[k], gsem[k])
    pltpu.sync_copy(zeros_hbm, acc.at[pl.ds(s * RPT, RPT)])
    plsc.subcore_barrier()

    for pnum, (off, rows) in enumerate(PHASES):
        if pnum > 0:
            base = s * CPT + off
            pltpu.sync_copy(sidx_hbm.at[c, pl.ds(base, rows)],
                            sidx_v.at[pl.ds(0, rows)])
            pltpu.sync_copy(didx_hbm.at[pl.ds(base, rows)],
                            didx_v.at[pl.ds(0, rows)])
            for k in range(NBUF - 1):
                pltpu.async_copy(t_hbm.at[sidx_v.at[k]], bufs[k], gsem[k])

        def body(jj, carry):
            for k in range(NBUF):
                i = NBUF * jj + k
                kp = (k + NBUF - 1) % NBUF

                @pl.when(i + NBUF - 1 < rows)
                def _():
                    @pl.when(i >= 1)
                    def _():
                        pltpu.make_async_copy(
                            bufs[kp], acc.at[didx_v.at[0]], tsem[kp]).wait()
                    pltpu.async_copy(t_hbm.at[sidx_v.at[i + NBUF - 1]],
                                     bufs[kp], gsem[kp])

                pltpu.make_async_copy(t_hbm.at[sidx_v.at[i]], bufs[k],
                                      gsem[k]).wait()
                pltpu.async_copy(bufs[k], acc.at[didx_v.at[i]], tsem[k],
                                 add=True)
            return carry

        lax.fori_loop(0, rows // NBUF, body, 0)
        for k in range(NBUF):
            pltpu.make_async_copy(bufs[k], acc.at[didx_v.at[0]],
                                  tsem[k]).wait()
    plsc.subcore_barrier()
    pltpu.sync_copy(acc.at[pl.ds(s * RPT, RPT)],
                    out_hbm.at[c, pl.ds(s * RPT, RPT)])


_agg_call = pl.kernel(
    _agg_body,
    out_type=jax.ShapeDtypeStruct((NC, NPAD, 128), jnp.float32),
    mesh=_mesh,
    scratch_types=[
        pltpu.VMEM((CPH, CW), jnp.int32),
        pltpu.VMEM((CPH, CW), jnp.int32),
        tuple(pltpu.VMEM((CW, 128), jnp.float32) for _ in range(NBUF)),
        pltpu.VMEM_SHARED((NPAD, 128), jnp.float32),
        tuple(pltpu.SemaphoreType.DMA for _ in range(NBUF)),
        tuple(pltpu.SemaphoreType.DMA for _ in range(NBUF)),
    ],
)


def _dinv_of(deg_ref):
    d = deg_ref[0].reshape(BN) + deg_ref[1].reshape(BN) + 1.0
    return lax.rsqrt(d)[:, None]


def _scale_kernel(x_ref, deg_ref, o_ref):
    dinv = _dinv_of(deg_ref)
    o_ref[0] = x_ref[:, :128] * dinv
    o_ref[1] = x_ref[:, 128:] * dinv


def _combine(agg_ref, t_ref, dinv):
    return jnp.concatenate([agg_ref[0] + t_ref[0], agg_ref[1] + t_ref[1]],
                           axis=1) * dinv


def _mm_kernel(agg_ref, t_ref, deg_ref, w1_ref, b1_ref, w2_ref, o_ref):
    dinv = _dinv_of(deg_ref)
    y = _combine(agg_ref, t_ref, dinv)
    h = jnp.dot(y, w1_ref[...], preferred_element_type=jnp.float32) + b1_ref[...]
    h = jnp.maximum(h, 0.0)
    q = jnp.dot(h, w2_ref[...], preferred_element_type=jnp.float32)
    q = q * dinv
    o_ref[0] = q[:, :128]
    o_ref[1] = q[:, 128:]


def _post_kernel(agg_ref, t_ref, deg_ref, b_ref, o_ref):
    dinv = _dinv_of(deg_ref)
    o_ref[...] = _combine(agg_ref, t_ref, dinv) + b_ref[...]


def _split_spec():
    return pl.BlockSpec((NC, BN, 128), lambda i: (0, i, 0))


def _deg_spec():
    return pl.BlockSpec((NC, BN // 128, 128), lambda i: (0, i, 0))


def _full(shape):
    return pl.BlockSpec(shape, lambda i: tuple(0 for _ in shape))


_scale_call = pl.pallas_call(
    _scale_kernel,
    grid=GRID,
    in_specs=[pl.BlockSpec((BN, 256), lambda i: (i, 0)), _deg_spec()],
    out_specs=_split_spec(),
    out_shape=jax.ShapeDtypeStruct((NC, NPAD, 128), jnp.float32),
)

_mm_call = pl.pallas_call(
    _mm_kernel,
    grid=GRID,
    in_specs=[_split_spec(), _split_spec(), _deg_spec(),
              _full((256, 512)), _full((1, 512)), _full((512, 256))],
    out_specs=_split_spec(),
    out_shape=jax.ShapeDtypeStruct((NC, NPAD, 128), jnp.float32),
)

_post_call = pl.pallas_call(
    _post_kernel,
    grid=GRID,
    in_specs=[_split_spec(), _split_spec(), _deg_spec(), _full((1, 256))],
    out_specs=pl.BlockSpec((BN, 256), lambda i: (i, 0)),
    out_shape=jax.ShapeDtypeStruct((N, 256), jnp.float32),
)


def kernel(x, edge_index, W1, b1, W2, b2):
    ei = edge_index.astype(jnp.int32)
    src = ei[0].reshape(E // CW, CW)
    src_stack = jnp.stack([src, src + NPAD])
    dst2d = ei[1].reshape(E // CW, CW)
    dstdeg = ei[1].reshape(E // EW, EW)
    ones_w = jnp.ones((EW,), jnp.float32)
    zeros1d = jnp.zeros((RPT,), jnp.float32)
    zeros2d = jnp.zeros((RPT, 128), jnp.float32)

    deg2 = _deg_call(dstdeg, ones_w, zeros1d)
    deg2p = deg2.reshape(NC, NPAD // 128, 128)

    t1 = _scale_call(x, deg2p)
    agg1 = _agg_call(t1.reshape(NC * NPAD, 128), src_stack, dst2d, zeros2d)
    t2 = _mm_call(agg1, t1, deg2p, W1, b1.reshape(1, -1), W2)
    agg2 = _agg_call(t2.reshape(NC * NPAD, 128), src_stack, dst2d, zeros2d)
    return _post_call(agg2, t2, deg2p, b2.reshape(1, -1))
